# Initial kernel scaffold; baseline (speedup 1.0000x reference)
import jax
import jax.numpy as jnp
from jax.experimental import pallas as pl


def kernel(hidden_states, router_weight, router_bias, gate_up_proj, gate_up_proj_bias, down_proj, down_proj_bias):
    raise NotImplementedError("write your pallas kernel here")


# speedup vs baseline: 7.0919x; 7.0919x over previous
import functools

import jax
import jax.numpy as jnp
import numpy as np
from jax.experimental import pallas as pl
from jax.experimental.pallas import tpu as pltpu

E = 64
K = 2
H = 768
INTER = 768
T = 64
F2 = 2 * INTER

_BIG = 1e30


@functools.cache
def _compress_matrix():
    s = np.zeros((F2, INTER), dtype=np.float32)
    s[2 * np.arange(INTER), np.arange(INTER)] = 1.0
    return jnp.asarray(s)


def _fiota(shape, dim):
    return jax.lax.broadcasted_iota(jnp.int32, shape, dim).astype(jnp.float32)


def _routing_body(x_ref, rw_ref, rb_ref, scores_ref, elist_ref):
    x = x_ref[...]
    rw = rw_ref[...]
    logits = jax.lax.dot_general(
        x, rw, (((1,), (1,)), ((), ())), preferred_element_type=jnp.float32)
    logits = logits + rb_ref[...]

    lane = jax.lax.broadcasted_iota(jnp.int32, (T, E), 1)
    lane_f = lane.astype(jnp.float32)

    m1 = jnp.max(logits, axis=1, keepdims=True)
    i1 = jnp.min(jnp.where(logits == m1, lane_f, _BIG), axis=1, keepdims=True)
    sel1 = lane_f == i1
    masked = jnp.where(sel1, -_BIG, logits)
    m2 = jnp.max(masked, axis=1, keepdims=True)
    i2 = jnp.min(jnp.where(masked == m2, lane_f, _BIG), axis=1, keepdims=True)
    sel2 = lane_f == i2

    e2 = jnp.exp(m2 - m1)
    w1 = 1.0 / (1.0 + e2)
    w2 = e2 / (1.0 + e2)
    scores = jnp.where(sel1, w1, 0.0) + jnp.where(sel2, w2, 0.0)
    scores_ref[...] = scores

    selected = jnp.logical_or(sel1, sel2).astype(jnp.float32)
    count = jnp.sum(selected, axis=0, keepdims=True)
    a = (count > 0.0).astype(jnp.float32)

    r = _fiota((E, E), 0)
    c = _fiota((E, E), 1)
    lower = (r <= c).astype(jnp.float32)
    cum = jax.lax.dot_general(
        a, lower, (((1,), (0,)), ((), ())), preferred_element_type=jnp.float32)
    nact = cum[:, E - 1:E]
    cum_i = jax.lax.dot_general(
        1.0 - a, lower, (((1,), (0,)), ((), ())),
        preferred_element_type=jnp.float32)
    pos = jnp.where(a > 0.0, cum - 1.0, nact + cum_i - 1.0)

    ident = (r == c).astype(jnp.float32)
    pos_col = jax.lax.dot_general(
        ident, pos, (((1,), (1,)), ((), ())), preferred_element_type=jnp.float32)
    a_col = jax.lax.dot_general(
        ident, a, (((1,), (1,)), ((), ())), preferred_element_type=jnp.float32)

    e_row = _fiota((1, E), 1)
    last_active = jnp.sum(
        jnp.where(jnp.logical_and(a > 0.0, pos == nact - 1.0), e_row, 0.0),
        axis=1, keepdims=True)

    e_sub = _fiota((E, E), 0)
    p_lane = _fiota((E, E), 1)
    ind = jnp.logical_and(pos_col == p_lane, a_col > 0.0)
    elist_active = jnp.sum(jnp.where(ind, e_sub, 0.0), axis=0, keepdims=True)
    p_row = _fiota((1, E), 1)
    elist = jnp.where(p_row < nact, elist_active, last_active)
    elist_ref[...] = jnp.broadcast_to(elist, (8, E)).astype(jnp.int32)


def _expert_body(el_ref, x_ref, sc_ref, wgu_ref, bgu_ref, wd_ref, bd_ref,
                 s_ref, out_ref):
    i = pl.program_id(0)
    e = el_ref[i]
    prev = el_ref[jnp.maximum(i - 1, 0)]
    valid = jnp.logical_or(i == 0, e != prev)

    @pl.when(valid)
    def _():
        x = x_ref[...]
        gu = jax.lax.dot_general(
            x, wgu_ref[0], (((1,), (0,)), ((), ())),
            preferred_element_type=jnp.float32)
        gu = gu + bgu_ref[0]
        lane = jax.lax.broadcasted_iota(jnp.int32, (T, F2), 1)
        even = (lane & 1) == 0
        up_sh = pltpu.roll(gu, F2 - 1, 1)
        gate = jnp.minimum(gu, 7.0)
        up = jnp.clip(up_sh, -7.0, 7.0)
        glu = gate * jax.nn.sigmoid(gate * 1.702)
        act2 = jnp.where(even, (up + 1.0) * glu, 0.0)
        act = jax.lax.dot_general(
            act2, s_ref[...], (((1,), (0,)), ((), ())),
            preferred_element_type=jnp.float32)
        oute = jax.lax.dot_general(
            act, wd_ref[0], (((1,), (0,)), ((), ())),
            preferred_element_type=jnp.float32)
        oute = oute + bd_ref[0]
        onehot = (jax.lax.broadcasted_iota(jnp.int32, (E, 1), 0) == e
                  ).astype(jnp.float32)
        col = jax.lax.dot_general(
            sc_ref[...], onehot, (((1,), (0,)), ((), ())),
            preferred_element_type=jnp.float32)
        contrib = oute * col

        @pl.when(i == 0)
        def _():
            out_ref[...] = contrib

        @pl.when(i > 0)
        def _():
            out_ref[...] += contrib


@jax.jit
def kernel(hidden_states, router_weight, router_bias, gate_up_proj,
           gate_up_proj_bias, down_proj, down_proj_bias):
    b, s, h = hidden_states.shape
    x = hidden_states.reshape(T, H)

    scores, elist8 = pl.pallas_call(
        _routing_body,
        out_shape=(
            jax.ShapeDtypeStruct((T, E), jnp.float32),
            jax.ShapeDtypeStruct((8, E), jnp.int32),
        ),
    )(x, router_weight, router_bias.reshape(1, E))
    elist = elist8[0]

    grid_spec = pltpu.PrefetchScalarGridSpec(
        num_scalar_prefetch=1,
        grid=(E,),
        in_specs=[
            pl.BlockSpec((T, H), lambda i, el: (0, 0)),
            pl.BlockSpec((T, E), lambda i, el: (0, 0)),
            pl.BlockSpec((1, H, F2), lambda i, el: (el[i], 0, 0)),
            pl.BlockSpec((1, 1, F2), lambda i, el: (el[i], 0, 0)),
            pl.BlockSpec((1, INTER, H), lambda i, el: (el[i], 0, 0)),
            pl.BlockSpec((1, 1, H), lambda i, el: (el[i], 0, 0)),
            pl.BlockSpec((F2, INTER), lambda i, el: (0, 0)),
        ],
        out_specs=pl.BlockSpec((T, H), lambda i, el: (0, 0)),
    )
    out = pl.pallas_call(
        _expert_body,
        grid_spec=grid_spec,
        out_shape=jax.ShapeDtypeStruct((T, H), jnp.float32),
        compiler_params=pltpu.CompilerParams(
            dimension_semantics=("arbitrary",)),
    )(elist, x, scores, gate_up_proj,
      gate_up_proj_bias.reshape(E, 1, F2), down_proj,
      down_proj_bias.reshape(E, 1, H), _compress_matrix())

    return out.reshape(b, s, h), scores
